# Initial kernel scaffold; baseline (speedup 1.0000x reference)
import jax
import jax.numpy as jnp
from jax.experimental import pallas as pl


def kernel(yday, embedding):
    raise NotImplementedError("write your pallas kernel here")


# speedup vs baseline: 6.3091x; 6.3091x over previous
import functools

import jax
import jax.numpy as jnp
from jax import lax
from jax.experimental import pallas as pl
from jax.experimental.pallas import tpu as pltpu
from jax.experimental.pallas import tpu_sc as plsc

NUM_NODES = 366
NUM_CHANNELS = 16

NC, NS, L = 2, 16, 16
NW = NC * NS
B = 16384 * 50
ROWS_PER_W = B // NW
CHUNK = 2560
NCHUNK = ROWS_PER_W // CHUNK
GROUPS = CHUNK // L


def _sc_body(y_hbm, tab_hbm, out_hbm, tab_v, y_v, out_v):
    wid = lax.axis_index("c") * NS + lax.axis_index("s")
    pltpu.sync_copy(tab_hbm, tab_v)

    lane = lax.broadcasted_iota(jnp.int32, (L,), 0)
    lane16 = lane * NUM_CHANNELS

    def group_body(g, _):
        yv = y_v[pl.ds(g * L, L)]
        scaled = yv * jnp.float32(NUM_NODES)
        l = scaled.astype(jnp.int32)
        alpha = scaled - l.astype(jnp.float32)
        i1 = l * 32
        sidx = lane16 + g * (L * NUM_CHANNELS)
        for c in range(NUM_CHANNELS):
            a = plsc.load_gather(tab_v, [i1 + c])
            b = plsc.load_gather(tab_v, [i1 + (NUM_CHANNELS + c)])
            plsc.store_scatter(out_v, [sidx + c], a + alpha * b)
        return 0

    for chunk in range(NCHUNK):
        base = wid * ROWS_PER_W + chunk * CHUNK
        pltpu.sync_copy(y_hbm.at[pl.ds(base, CHUNK)], y_v)
        lax.fori_loop(0, GROUPS, group_body, 0)
        pltpu.sync_copy(out_v, out_hbm.at[pl.ds(base * NUM_CHANNELS,
                                                CHUNK * NUM_CHANNELS)])


@jax.jit
def _run(y_flat, comb_flat):
    mesh = plsc.VectorSubcoreMesh(core_axis_name="c", subcore_axis_name="s")
    f = pl.kernel(
        _sc_body,
        out_type=jax.ShapeDtypeStruct((B * NUM_CHANNELS,), jnp.float32),
        mesh=mesh,
        scratch_types=[
            pltpu.VMEM((NUM_NODES * 32,), jnp.float32),
            pltpu.VMEM((CHUNK,), jnp.float32),
            pltpu.VMEM((CHUNK * NUM_CHANNELS,), jnp.float32),
        ],
        compiler_params=pltpu.CompilerParams(needs_layout_passes=False),
    )
    return f(y_flat, comb_flat)


def kernel(yday, embedding):
    upper = jnp.roll(embedding, -1, axis=0)
    comb = jnp.concatenate([upper, embedding - upper], axis=1)
    out = _run(yday.reshape(-1), comb.reshape(-1))
    return out.reshape(B, NUM_CHANNELS)


# speedup vs baseline: 7.6223x; 1.2081x over previous
import functools

import jax
import jax.numpy as jnp
from jax import lax
from jax.experimental import pallas as pl
from jax.experimental.pallas import tpu as pltpu
from jax.experimental.pallas import tpu_sc as plsc

NUM_NODES = 366
NUM_CHANNELS = 16

NC, NS, L = 2, 16, 16
NW = NC * NS
B = 16384 * 50
ROWS_PER_W = B // NW
CHUNK = 2560
NCHUNK = ROWS_PER_W // CHUNK
GROUPS = CHUNK // L


def _sc_body(y_hbm, tab_hbm, out_hbm, tab_v, y_v, out_v):
    wid = lax.axis_index("c") * NS + lax.axis_index("s")
    pltpu.sync_copy(tab_hbm, tab_v)

    lane = lax.broadcasted_iota(jnp.int32, (L,), 0)
    lane16 = lane * NUM_CHANNELS

    def group_body(g, _):
        scaled = y_v[pl.ds(g * L, L)]
        l = scaled.astype(jnp.int32)
        alpha = scaled - l.astype(jnp.float32)
        i1 = l * 32
        sidx = lane16 + g * (L * NUM_CHANNELS)
        ga = [plsc.load_gather(tab_v, [i1 + c]) for c in range(NUM_CHANNELS)]
        gb = [plsc.load_gather(tab_v, [i1 + (NUM_CHANNELS + c)])
              for c in range(NUM_CHANNELS)]
        vals = [ga[c] + alpha * gb[c] for c in range(NUM_CHANNELS)]
        for c in range(NUM_CHANNELS):
            plsc.store_scatter(out_v, [sidx + c], vals[c])
        return 0

    for chunk in range(NCHUNK):
        base = wid * ROWS_PER_W + chunk * CHUNK
        pltpu.sync_copy(y_hbm.at[pl.ds(base, CHUNK)], y_v)
        lax.fori_loop(0, GROUPS, group_body, 0)
        pltpu.sync_copy(out_v, out_hbm.at[pl.ds(base * NUM_CHANNELS,
                                                CHUNK * NUM_CHANNELS)])


@jax.jit
def _run(y_flat, comb_flat):
    mesh = plsc.VectorSubcoreMesh(core_axis_name="c", subcore_axis_name="s")
    f = pl.kernel(
        _sc_body,
        out_type=jax.ShapeDtypeStruct((B * NUM_CHANNELS,), jnp.float32),
        mesh=mesh,
        scratch_types=[
            pltpu.VMEM((NUM_NODES * 32,), jnp.float32),
            pltpu.VMEM((CHUNK,), jnp.float32),
            pltpu.VMEM((CHUNK * NUM_CHANNELS,), jnp.float32),
        ],
        compiler_params=pltpu.CompilerParams(needs_layout_passes=False),
    )
    return f(y_flat, comb_flat)


def kernel(yday, embedding):
    upper = jnp.roll(embedding, -1, axis=0)
    comb = jnp.concatenate([upper, embedding - upper], axis=1)
    scaled = (yday * jnp.float32(NUM_NODES)).reshape(-1)
    out = _run(scaled, comb.reshape(-1))
    return out.reshape(B, NUM_CHANNELS)


# speedup vs baseline: 47.7507x; 6.2646x over previous
import jax
import jax.numpy as jnp
from jax import lax
from jax.experimental import pallas as pl
from jax.experimental.pallas import tpu as pltpu
from jax.experimental.pallas import tpu_sc as plsc

NUM_NODES = 366
NUM_CHANNELS = 16
STRIDE = 33
TAB_WORDS = 12080

NC, NS, L = 2, 16, 16
NW = NC * NS
B = 16384 * 50
ROWS_PER_W = B // NW
CHUNK = 2560
NCHUNK = ROWS_PER_W // CHUNK
GROUPS = CHUNK // L


def _sc_body(y_hbm, tab_hbm, out_hbm, tab_v, y_v, out_v):
    wid = lax.axis_index("c") * NS + lax.axis_index("s")
    pltpu.sync_copy(tab_hbm, tab_v)

    def group_body(g, _):
        scaled = y_v[pl.ds(g * L, L)]
        li = scaled.astype(jnp.int32)
        alpha = scaled - li.astype(jnp.float32)
        i1 = li * STRIDE
        r0 = g * L
        ga = [plsc.load_gather(tab_v, [i1 + c]) for c in range(NUM_CHANNELS)]
        gb = [plsc.load_gather(tab_v, [i1 + (NUM_CHANNELS + c)])
              for c in range(NUM_CHANNELS)]
        for c in range(NUM_CHANNELS):
            out_v[c, pl.ds(r0, L)] = ga[c] + alpha * gb[c]
        return 0

    def chunk_body(chunk, _):
        base = wid * ROWS_PER_W + chunk * CHUNK
        pltpu.sync_copy(y_hbm.at[pl.ds(base, CHUNK)], y_v)
        lax.fori_loop(0, GROUPS, group_body, 0)
        pltpu.sync_copy(out_v, out_hbm.at[:, pl.ds(base, CHUNK)])
        return 0

    lax.fori_loop(0, NCHUNK, chunk_body, 0)


@jax.jit
def _run(y_flat, comb_flat):
    mesh = plsc.VectorSubcoreMesh(core_axis_name="c", subcore_axis_name="s")
    f = pl.kernel(
        _sc_body,
        out_type=jax.ShapeDtypeStruct((NUM_CHANNELS, B), jnp.float32),
        mesh=mesh,
        scratch_types=[
            pltpu.VMEM((TAB_WORDS,), jnp.float32),
            pltpu.VMEM((CHUNK,), jnp.float32),
            pltpu.VMEM((NUM_CHANNELS, CHUNK), jnp.float32),
        ],
        compiler_params=pltpu.CompilerParams(needs_layout_passes=False),
    )
    return f(y_flat, comb_flat)


def kernel(yday, embedding):
    upper = jnp.roll(embedding, -1, axis=0)
    comb = jnp.concatenate(
        [upper, embedding - upper,
         jnp.zeros((NUM_NODES, STRIDE - 2 * NUM_CHANNELS), jnp.float32)],
        axis=1)
    comb_flat = jnp.pad(comb.reshape(-1), (0, TAB_WORDS - NUM_NODES * STRIDE))
    scaled = (yday * jnp.float32(NUM_NODES)).reshape(-1)
    out_t = _run(scaled, comb_flat)
    return out_t.T


# speedup vs baseline: 57.8243x; 1.2110x over previous
import jax
import jax.numpy as jnp
from jax import lax
from jax.experimental import pallas as pl
from jax.experimental.pallas import tpu as pltpu
from jax.experimental.pallas import tpu_sc as plsc

NUM_NODES = 366
NUM_CHANNELS = 16
STRIDE = 33
TAB_WORDS = 12080

NC, NS, L = 2, 16, 16
NW = NC * NS
B = 16384 * 50
ROWS_PER_W = B // NW
CHUNK = 2560
NCHUNK = ROWS_PER_W // CHUNK
GROUPS = CHUNK // L


def _sc_body(y_hbm, tab_hbm, out_hbm, tab_v,
             y_v0, y_v1, out_v0, out_v1,
             in_sem0, in_sem1, out_sem0, out_sem1):
    wid = lax.axis_index("c") * NS + lax.axis_index("s")
    row0 = wid * ROWS_PER_W
    y_bufs = (y_v0, y_v1)
    out_bufs = (out_v0, out_v1)
    in_sems = (in_sem0, in_sem1)
    out_sems = (out_sem0, out_sem1)
    pltpu.sync_copy(tab_hbm, tab_v)

    def in_copy(k, b):
        return pltpu.make_async_copy(
            y_hbm.at[pl.ds(row0 + k * CHUNK, CHUNK)], y_bufs[b], in_sems[b])

    def out_copy(k, b):
        return pltpu.make_async_copy(
            out_bufs[b], out_hbm.at[:, pl.ds(row0 + k * CHUNK, CHUNK)],
            out_sems[b])

    def make_group_body(y_v, out_v):
        def group_body(g, _):
            scaled = y_v[pl.ds(g * L, L)]
            li = scaled.astype(jnp.int32)
            alpha = scaled - li.astype(jnp.float32)
            i1 = li * STRIDE
            r0 = g * L
            ga = [plsc.load_gather(tab_v, [i1 + c])
                  for c in range(NUM_CHANNELS)]
            gb = [plsc.load_gather(tab_v, [i1 + (NUM_CHANNELS + c)])
                  for c in range(NUM_CHANNELS)]
            for c in range(NUM_CHANNELS):
                out_v[c, pl.ds(r0, L)] = ga[c] + alpha * gb[c]
            return 0
        return group_body

    in_copy(0, 0).start()
    for k in range(NCHUNK):
        b = k % 2
        if k + 1 < NCHUNK:
            in_copy(k + 1, 1 - b).start()
        in_copy(k, b).wait()
        if k >= 2:
            out_copy(k - 2, b).wait()
        lax.fori_loop(0, GROUPS, make_group_body(y_bufs[b], out_bufs[b]), 0)
        out_copy(k, b).start()
    out_copy(NCHUNK - 2, NCHUNK % 2).wait()
    out_copy(NCHUNK - 1, (NCHUNK - 1) % 2).wait()


@jax.jit
def _run(y_flat, comb_flat):
    mesh = plsc.VectorSubcoreMesh(core_axis_name="c", subcore_axis_name="s")
    f = pl.kernel(
        _sc_body,
        out_type=jax.ShapeDtypeStruct((NUM_CHANNELS, B), jnp.float32),
        mesh=mesh,
        scratch_types=[
            pltpu.VMEM((TAB_WORDS,), jnp.float32),
            pltpu.VMEM((CHUNK,), jnp.float32),
            pltpu.VMEM((CHUNK,), jnp.float32),
            pltpu.VMEM((NUM_CHANNELS, CHUNK), jnp.float32),
            pltpu.VMEM((NUM_CHANNELS, CHUNK), jnp.float32),
            pltpu.SemaphoreType.DMA,
            pltpu.SemaphoreType.DMA,
            pltpu.SemaphoreType.DMA,
            pltpu.SemaphoreType.DMA,
        ],
        compiler_params=pltpu.CompilerParams(needs_layout_passes=False),
    )
    return f(y_flat, comb_flat)


def kernel(yday, embedding):
    upper = jnp.roll(embedding, -1, axis=0)
    comb = jnp.concatenate(
        [upper, embedding - upper,
         jnp.zeros((NUM_NODES, STRIDE - 2 * NUM_CHANNELS), jnp.float32)],
        axis=1)
    comb_flat = jnp.pad(comb.reshape(-1), (0, TAB_WORDS - NUM_NODES * STRIDE))
    scaled = (yday * jnp.float32(NUM_NODES)).reshape(-1)
    out_t = _run(scaled, comb_flat)
    return out_t.T
